# Initial kernel scaffold; baseline (speedup 1.0000x reference)
import jax
import jax.numpy as jnp
from jax.experimental import pallas as pl


def kernel(token_ids, activity_ids, resource_ids, numeric_features, time_features, token_table, activity_table, resource_table, num_W1, num_b1, num_W2, num_b2, time_W1, time_b1, time_W2, time_b2, proj_W, proj_b):
    raise NotImplementedError("write your pallas kernel here")


# speedup vs baseline: 1.2518x; 1.2518x over previous
import functools

import numpy as np
import jax
import jax.numpy as jnp
from jax import lax
from jax.experimental import pallas as pl
from jax.experimental.pallas import tpu as pltpu
from jax.experimental.pallas import tpu_sc as plsc

B, S, V, D = 1024, 50, 100000, 128
N = B * S

_NC, _NS = 2, 16
NW = _NC * _NS
PER_W = N // NW
CH = 128
NFULL = PER_W // CH
TAIL = PER_W - NFULL * CH


def _make_pe():
    position = np.arange(S)[:, None].astype(np.float64)
    div_term = np.exp(np.arange(0, D, 2).astype(np.float64) * (-np.log(10000.0) / D))
    pe = np.zeros((S, D), dtype=np.float32)
    pe[:, 0::2] = np.sin(position * div_term)
    pe[:, 1::2] = np.cos(position * div_term)
    return pe


def _sc_gather3(tok_tab, act_tab, res_tab, tok_idx, act_idx, res_idx):
    mesh = plsc.VectorSubcoreMesh(
        core_axis_name="c", subcore_axis_name="s",
        num_cores=_NC, num_subcores=_NS)

    @functools.partial(
        pl.kernel,
        out_type=(jax.ShapeDtypeStruct((N, D), jnp.float32),) * 3,
        mesh=mesh,
        scratch_types=[
            pltpu.VMEM((PER_W,), jnp.int32),
            pltpu.VMEM((CH, D), jnp.float32),
            pltpu.SemaphoreType.DMA,
        ],
    )
    def gather_k(tok_tab, act_tab, res_tab, tok_i, act_i, res_i,
                 o_tok, o_act, o_res, idx_v, rows_v, sem):
        wid = lax.axis_index("s") * _NC + lax.axis_index("c")
        base = wid * PER_W
        for tab, idx_hbm, out_hbm in ((tok_tab, tok_i, o_tok),
                                      (act_tab, act_i, o_act),
                                      (res_tab, res_i, o_res)):
            pltpu.sync_copy(idx_hbm.at[pl.ds(base, PER_W)], idx_v)

            def chunk(c, carry, tab=tab, out_hbm=out_hbm):
                row0 = c * CH
                pltpu.async_copy(
                    tab.at[idx_v.at[pl.ds(row0, CH)]], rows_v, sem).wait()
                pltpu.sync_copy(rows_v, out_hbm.at[pl.ds(base + row0, CH)])
                return carry

            lax.fori_loop(0, NFULL, chunk, 0)
            row0 = NFULL * CH
            pltpu.async_copy(
                tab.at[idx_v.at[pl.ds(row0, TAIL)]],
                rows_v.at[pl.ds(0, TAIL)], sem).wait()
            pltpu.sync_copy(rows_v.at[pl.ds(0, TAIL)],
                            out_hbm.at[pl.ds(base + row0, TAIL)])

    return gather_k(tok_tab, act_tab, res_tab, tok_idx, act_idx, res_idx)


NR = 512


def _tc_body(aux_ref, tok_ref, act_ref, res_ref, nW1, nb1, nW2, nb2,
             tW1, tb1, tW2, tb2, pW, pe_ref, out_ref):
    a = aux_ref[...]
    m = a[:, 0:1]
    nf = a[:, 1:2]
    t0 = a[:, 2:3]
    t1 = a[:, 3:4]
    h_n = jnp.maximum(nf * nW1[0:1, :] + nb1[0:1, :], 0.0)
    v_n = jnp.dot(h_n, nW2[...], preferred_element_type=jnp.float32) + nb2[0:1, :]
    h_t = jnp.maximum(t0 * tW1[0:1, :] + t1 * tW1[1:2, :] + tb1[0:1, :], 0.0)
    v_t = jnp.dot(h_t, tW2[...], preferred_element_type=jnp.float32) + tb2[0:1, :]
    w = pW[...]
    p = (jnp.dot(act_ref[...], w[0:D, :], preferred_element_type=jnp.float32)
         + jnp.dot(res_ref[...], w[D:2 * D, :], preferred_element_type=jnp.float32)
         + jnp.dot(v_n, w[2 * D:3 * D, :], preferred_element_type=jnp.float32)
         + jnp.dot(v_t, w[3 * D:4 * D, :], preferred_element_type=jnp.float32))
    out_ref[...] = m * p + tok_ref[...] + pe_ref[0]


def _tc_fuse(aux, tok_rows, act_rows, res_rows,
             num_W1, num_b1, num_W2, num_b2,
             time_W1, time_b1, time_W2, time_b2, proj_W, pe_pb):
    rows_spec = pl.BlockSpec((NR, D), lambda i: (i, 0))
    full = lambda shape: pl.BlockSpec(shape, lambda i: (0,) * len(shape))
    return pl.pallas_call(
        _tc_body,
        grid=(N // NR,),
        in_specs=[
            pl.BlockSpec((NR, 4), lambda i: (i, 0)),
            rows_spec, rows_spec, rows_spec,
            full((1, D // 2)), full((1, D // 2)),
            full((D // 2, D)), full((1, D)),
            full((2, D // 2)), full((1, D // 2)),
            full((D // 2, D)), full((1, D)),
            full((4 * D, D)),
            pl.BlockSpec((1, 1, D), lambda i: (i // (B // NR), 0, 0)),
        ],
        out_specs=pl.BlockSpec((NR, D), lambda i: (i, 0)),
        out_shape=jax.ShapeDtypeStruct((N, D), jnp.float32),
    )(aux, tok_rows, act_rows, res_rows,
      num_W1, num_b1, num_W2, num_b2,
      time_W1, time_b1, time_W2, time_b2, proj_W, pe_pb)


def kernel(token_ids, activity_ids, resource_ids, numeric_features, time_features,
           token_table, activity_table, resource_table,
           num_W1, num_b1, num_W2, num_b2,
           time_W1, time_b1, time_W2, time_b2,
           proj_W, proj_b):
    tok_idx = token_ids.T.reshape(N).astype(jnp.int32)
    act_idx = activity_ids.T.reshape(N).astype(jnp.int32)
    res_idx = resource_ids.T.reshape(N).astype(jnp.int32)
    mask = (activity_ids.T > 0).astype(jnp.float32)[..., None]
    numT = numeric_features.transpose(1, 0, 2)
    timeT = time_features.transpose(1, 0, 2)
    aux = jnp.concatenate([mask, numT, timeT], axis=-1).reshape(N, 4)

    tok_rows, act_rows, res_rows = _sc_gather3(
        token_table, activity_table, resource_table, tok_idx, act_idx, res_idx)

    pe_pb = (jnp.asarray(_make_pe()) + proj_b[None, :]).reshape(S, 1, D)
    out = _tc_fuse(aux, tok_rows, act_rows, res_rows,
                   num_W1.reshape(1, D // 2), num_b1.reshape(1, D // 2),
                   num_W2, num_b2.reshape(1, D),
                   time_W1, time_b1.reshape(1, D // 2),
                   time_W2, time_b2.reshape(1, D),
                   proj_W, pe_pb)
    return out.reshape(S, B, D)
